# baseline (speedup 1.0000x reference)
import jax
import jax.numpy as jnp
from jax import lax
from jax.experimental import pallas as pl
from jax.experimental.pallas import tpu as pltpu
from jax.experimental.pallas import tpu_sc as plsc

B = 16384
D = 32
NC = 2
NS = 16
L = 16
NW = NC * NS
BPW = B // NW
RING = 8
BLK = 128


def _sc_body(uid_hbm, iid_hbm, ut_hbm, it_hbm, out_hbm,
             uidx_v, iidx_v, ub, ib, out_v, sems_u, sems_i):
    wid = lax.axis_index("s") * NC + lax.axis_index("c")
    base = wid * BPW
    pltpu.sync_copy(uid_hbm.at[pl.ds(base, BPW)], uidx_v)
    pltpu.sync_copy(iid_hbm.at[pl.ds(base, BPW)], iidx_v)

    lanes = lax.iota(jnp.int32, L)
    NG = BPW // L

    def issue(uid, iid, slot):
        ublk = pl.multiple_of(uid & ~(BLK - 1), BLK)
        iblk = pl.multiple_of(iid & ~(BLK - 1), BLK)
        pltpu.make_async_copy(
            ut_hbm.at[:, pl.ds(ublk, BLK)], ub.at[slot], sems_u.at[slot]
        ).start()
        pltpu.make_async_copy(
            it_hbm.at[:, pl.ds(iblk, BLK)], ib.at[slot], sems_i.at[slot]
        ).start()

    def wait(slot):
        pltpu.make_async_copy(
            ut_hbm.at[:, pl.ds(0, BLK)], ub.at[slot], sems_u.at[slot]
        ).wait()
        pltpu.make_async_copy(
            it_hbm.at[:, pl.ds(0, BLK)], ib.at[slot], sems_i.at[slot]
        ).wait()

    def extract_dot(uid, iid, slot):
        ucols = jnp.zeros((L,), jnp.int32) + (uid & (BLK - 1))
        icols = jnp.zeros((L,), jnp.int32) + (iid & (BLK - 1))
        u0 = plsc.load_gather(ub.at[slot], [lanes, ucols])
        u1 = plsc.load_gather(ub.at[slot], [lanes + L, ucols])
        i0 = plsc.load_gather(ib.at[slot], [lanes, icols])
        i1 = plsc.load_gather(ib.at[slot], [lanes + L, icols])
        return jnp.sum(u0 * i0 + u1 * i1)

    vu0 = uidx_v[pl.ds(0, L)]
    vi0 = iidx_v[pl.ds(0, L)]
    for s in range(RING):
        issue(vu0[s], vi0[s], s)

    def step(g, carry):
        vu = uidx_v[pl.ds(g * L, L)]
        vi = iidx_v[pl.ds(g * L, L)]
        nxt = jnp.minimum(g + 1, NG - 1) * L
        vun = uidx_v[pl.ds(nxt, L)]
        vin = iidx_v[pl.ds(nxt, L)]
        acc = jnp.zeros((L,), jnp.float32)
        for j in range(L):
            slot = j % RING
            wait(slot)
            dot = extract_dot(vu[j], vi[j], slot)
            if j < L - RING:
                issue(vu[j + RING], vi[j + RING], slot)
            else:

                @pl.when(g + 1 < NG)
                def _(_j=j):
                    issue(vun[_j + RING - L], vin[_j + RING - L], slot)

            acc = jnp.where(lanes == j, dot, acc)
        out_v[pl.ds(g * L, L)] = acc
        return carry

    lax.fori_loop(0, NG, step, 0)
    pltpu.sync_copy(out_v, out_hbm.at[pl.ds(base, BPW)])


@jax.jit
def kernel(user_ids, item_ids, user_table, item_table):
    mesh = plsc.VectorSubcoreMesh(core_axis_name="c", subcore_axis_name="s")
    f = pl.kernel(
        _sc_body,
        out_type=jax.ShapeDtypeStruct((B,), jnp.float32),
        mesh=mesh,
        compiler_params=pltpu.CompilerParams(needs_layout_passes=False),
        scratch_types=[
            pltpu.VMEM((BPW,), jnp.int32),
            pltpu.VMEM((BPW,), jnp.int32),
            pltpu.VMEM((RING, D, BLK), jnp.float32),
            pltpu.VMEM((RING, D, BLK), jnp.float32),
            pltpu.VMEM((BPW,), jnp.float32),
            pltpu.SemaphoreType.DMA((RING,)),
            pltpu.SemaphoreType.DMA((RING,)),
        ],
    )
    return f(user_ids, item_ids, user_table.T, item_table.T)
